# baseline (speedup 1.0000x reference)
import functools

import jax
import jax.numpy as jnp
from jax import lax
from jax.experimental import pallas as pl
from jax.experimental.pallas import tpu as pltpu
from jax.experimental.pallas import tpu_sc as plsc

NUM_CORES = 2
NUM_SUBCORES = 16
NW = NUM_CORES * NUM_SUBCORES
CHUNK = 128


@functools.cache
def _build(B, D):
    assert B % (NW * CHUNK) == 0
    steps = B // (NW * CHUNK)
    mesh = plsc.VectorSubcoreMesh(core_axis_name="c", subcore_axis_name="s")

    @functools.partial(
        pl.kernel,
        out_type=jax.ShapeDtypeStruct((B, D), jnp.float32),
        mesh=mesh,
        scratch_types=[
            pltpu.VMEM((steps, CHUNK), jnp.int32),
            pltpu.VMEM((CHUNK, D), jnp.float32),
            pltpu.SemaphoreType.DMA,
        ],
        compiler_params=pltpu.CompilerParams(use_tc_tiling_on_sc=False),
    )
    def _gather(idx_hbm, table_hbm, out_hbm, idx_v, rows_v, sem):
        wid = lax.axis_index("s") * NUM_CORES + lax.axis_index("c")
        row0 = wid * steps
        pltpu.sync_copy(idx_hbm.at[pl.ds(row0, steps)], idx_v)

        @pl.loop(0, steps)
        def _chunk(g):
            pltpu.async_copy(table_hbm.at[idx_v.at[g]], rows_v, sem).wait()
            pltpu.sync_copy(rows_v, out_hbm.at[pl.ds((row0 + g) * CHUNK, CHUNK)])

    return _gather


@jax.jit
def kernel(token_ids, weights):
    S, T = token_ids.shape
    B = S * T
    D = weights.shape[1]
    idx2d = token_ids.reshape(B // CHUNK, CHUNK).astype(jnp.int32)
    out = _build(B, D)(idx2d, weights)
    return out.reshape(S, T, D)


# speedup vs baseline: 1.0644x; 1.0644x over previous
import functools

import jax
import jax.numpy as jnp
from jax import lax
from jax.experimental import pallas as pl
from jax.experimental.pallas import tpu as pltpu
from jax.experimental.pallas import tpu_sc as plsc

NUM_CORES = 2
NUM_SUBCORES = 16
NW = NUM_CORES * NUM_SUBCORES
CHUNK = 128
K = 2
NBUF = 4
LEAD = 2


@functools.cache
def _build(B, D):
    assert B % (NW * CHUNK * K) == 0
    chunks_pw = B // (NW * CHUNK)
    ngroups = chunks_pw // K
    assert ngroups % NBUF == 0
    mesh = plsc.VectorSubcoreMesh(core_axis_name="c", subcore_axis_name="s")

    @functools.partial(
        pl.kernel,
        out_type=jax.ShapeDtypeStruct((B, D), jnp.float32),
        mesh=mesh,
        scratch_types=[
            pltpu.VMEM((chunks_pw, CHUNK), jnp.int32),
            pltpu.VMEM((NBUF, K * CHUNK, D), jnp.float32),
        ]
        + [pltpu.SemaphoreType.DMA] * (2 * NBUF),
        compiler_params=pltpu.CompilerParams(use_tc_tiling_on_sc=False),
    )
    def _gather(idx_hbm, table_hbm, out_hbm, idx_v, rows_v, *sems):
        semg, sems_st = sems[:NBUF], sems[NBUF:]
        wid = lax.axis_index("s") * NUM_CORES + lax.axis_index("c")
        chunk0 = wid * chunks_pw
        pltpu.sync_copy(idx_hbm.at[pl.ds(chunk0, chunks_pw)], idx_v)

        def gather_group(group, p, wait):
            for j in range(K):
                dma = (pltpu.make_async_copy if wait else pltpu.async_copy)(
                    table_hbm.at[idx_v.at[group * K + j]],
                    rows_v.at[p, pl.ds(j * CHUNK, CHUNK)],
                    semg[p],
                )
                if wait:
                    dma.wait()

        def store_group(group, p, wait):
            dma = (pltpu.make_async_copy if wait else pltpu.async_copy)(
                rows_v.at[p],
                out_hbm.at[pl.ds((chunk0 + group * K) * CHUNK, K * CHUNK)],
                sems_st[p],
            )
            if wait:
                dma.wait()

        for p in range(LEAD):
            gather_group(p, p, wait=False)

        @pl.loop(0, ngroups, step=NBUF)
        def _grp(g0):
            for p in range(NBUF):
                group = g0 + p
                gather_group(group, p, wait=True)
                store_group(group, p, wait=False)
                q = (p + LEAD) % NBUF

                @pl.when(group + LEAD < ngroups)
                def _issue():
                    @pl.when(group >= NBUF - LEAD)
                    def _drain_store():
                        store_group(group + LEAD - NBUF, q, wait=True)

                    gather_group(group + LEAD, q, wait=False)

        for p in range(NBUF):
            store_group(ngroups - NBUF + p, p, wait=True)

    return _gather


@jax.jit
def kernel(token_ids, weights):
    S, T = token_ids.shape
    B = S * T
    D = weights.shape[1]
    idx2d = token_ids.reshape(B // CHUNK, CHUNK).astype(jnp.int32)
    out = _build(B, D)(idx2d, weights)
    return out.reshape(S, T, D)
